# baseline (speedup 1.0000x reference)
import functools

import jax
import jax.numpy as jnp
from jax.experimental import pallas as pl

D_MODEL = 1024
D_SPACE = 64
N_F = 256
N_R = 128
N_V = 256
N_USED = N_F + N_R + N_V
TK_F = 8
TK_R = 4
TK_V = 6


def _route(logits, k):
    m = jnp.max(logits, axis=-1, keepdims=True)
    neg = jnp.float32(-jnp.inf)
    w = jnp.where(logits == m, neg, logits)
    for _ in range(k - 2):
        cm = jnp.max(w, axis=-1, keepdims=True)
        w = jnp.where(w == cm, neg, w)
    thr = jnp.max(w, axis=-1, keepdims=True)
    e = jnp.exp(logits - m)
    z = jnp.sum(e, axis=-1, keepdims=True)
    kept = jnp.where(logits >= thr, e, 0.0)
    s = jnp.sum(kept, axis=-1, keepdims=True)
    return kept * (1.0 / (s + 1e-8 * z))


def _block_kernel(x_ref, w_ref, b_ref, ne_ref, f_ref, r_ref, v_ref):
    x = x_ref[...]
    h = jnp.dot(x, w_ref[...], preferred_element_type=jnp.float32) + b_ref[...]
    ne = ne_ref[...]
    norm = jnp.sqrt(jnp.sum(ne * ne, axis=-1, keepdims=True))
    ne_n = ne / jnp.maximum(norm, 1e-12)
    logits = jax.lax.dot_general(
        h, ne_n, (((1,), (1,)), ((), ())), preferred_element_type=jnp.float32
    )
    f_ref[...] = _route(logits[:, :N_F], TK_F)
    r_ref[...] = _route(logits[:, N_F:N_F + N_R], TK_R)
    v_ref[...] = _route(logits[:, N_F + N_R:N_USED], TK_V)


@jax.jit
def kernel(x, importance, W_proj, b_proj, neuron_emb):
    del importance
    B, S, D = x.shape
    T = B * S
    xf = x.reshape(T, D)
    ne = neuron_emb[:N_USED]
    b2 = b_proj.reshape(1, D_SPACE)
    BT = 2048
    f, r, v = pl.pallas_call(
        _block_kernel,
        grid=(T // BT,),
        in_specs=[
            pl.BlockSpec((BT, D_MODEL), lambda i: (i, 0)),
            pl.BlockSpec((D_MODEL, D_SPACE), lambda i: (0, 0)),
            pl.BlockSpec((1, D_SPACE), lambda i: (0, 0)),
            pl.BlockSpec((N_USED, D_SPACE), lambda i: (0, 0)),
        ],
        out_specs=[
            pl.BlockSpec((BT, N_F), lambda i: (i, 0)),
            pl.BlockSpec((BT, N_R), lambda i: (i, 0)),
            pl.BlockSpec((BT, N_V), lambda i: (i, 0)),
        ],
        out_shape=[
            jax.ShapeDtypeStruct((T, N_F), jnp.float32),
            jax.ShapeDtypeStruct((T, N_R), jnp.float32),
            jax.ShapeDtypeStruct((T, N_V), jnp.float32),
        ],
    )(xf, W_proj, b2, ne)
    fw = f.reshape(B, S, N_F)
    rw = r.reshape(B, S, N_R)
    vw = v.reshape(B, S, N_V)
    return (fw, rw, rw, vw)


# speedup vs baseline: 1.0963x; 1.0963x over previous
import jax
import jax.numpy as jnp
from jax.experimental import pallas as pl

D_MODEL = 1024
D_SPACE = 64
N_F = 256
N_R = 128
N_V = 256
N_USED = N_F + N_R + N_V
TK_F = 8
TK_R = 4
TK_V = 6


def _route_t(lt, k):
    m = jnp.max(lt, axis=0, keepdims=True)
    neg = jnp.float32(-jnp.inf)
    w = jnp.where(lt == m, neg, lt)
    for _ in range(k - 2):
        cm = jnp.max(w, axis=0, keepdims=True)
        w = jnp.where(w == cm, neg, w)
    thr = jnp.max(w, axis=0, keepdims=True)
    e = jnp.exp(lt - m)
    z = jnp.sum(e, axis=0, keepdims=True)
    kept = jnp.where(lt >= thr, e, 0.0)
    s = jnp.sum(kept, axis=0, keepdims=True)
    out_t = kept * (1.0 / (s + 1e-8 * z))
    return jnp.transpose(out_t)


def _block_kernel(x_ref, w_ref, b_ref, ne_ref, f_ref, r_ref, v_ref):
    x = x_ref[...]
    ht = jax.lax.dot_general(
        w_ref[...], x, (((0,), (1,)), ((), ())), preferred_element_type=jnp.float32
    ) + jnp.transpose(b_ref[...])
    ne = ne_ref[...]
    norm = jnp.sqrt(jnp.sum(ne * ne, axis=-1, keepdims=True))
    ne_n = ne / jnp.maximum(norm, 1e-12)
    lt = jax.lax.dot_general(
        ne_n, ht, (((1,), (0,)), ((), ())), preferred_element_type=jnp.float32
    )
    f_ref[...] = _route_t(lt[:N_F], TK_F)
    r_ref[...] = _route_t(lt[N_F:N_F + N_R], TK_R)
    v_ref[...] = _route_t(lt[N_F + N_R:N_USED], TK_V)


@jax.jit
def kernel(x, importance, W_proj, b_proj, neuron_emb):
    del importance
    B, S, D = x.shape
    T = B * S
    xf = x.reshape(T, D)
    ne = neuron_emb[:N_USED]
    b2 = b_proj.reshape(1, D_SPACE)
    BT = 1024
    f, r, v = pl.pallas_call(
        _block_kernel,
        grid=(T // BT,),
        in_specs=[
            pl.BlockSpec((BT, D_MODEL), lambda i: (i, 0)),
            pl.BlockSpec((D_MODEL, D_SPACE), lambda i: (0, 0)),
            pl.BlockSpec((1, D_SPACE), lambda i: (0, 0)),
            pl.BlockSpec((N_USED, D_SPACE), lambda i: (0, 0)),
        ],
        out_specs=[
            pl.BlockSpec((BT, N_F), lambda i: (i, 0)),
            pl.BlockSpec((BT, N_R), lambda i: (i, 0)),
            pl.BlockSpec((BT, N_V), lambda i: (i, 0)),
        ],
        out_shape=[
            jax.ShapeDtypeStruct((T, N_F), jnp.float32),
            jax.ShapeDtypeStruct((T, N_R), jnp.float32),
            jax.ShapeDtypeStruct((T, N_V), jnp.float32),
        ],
    )(xf, W_proj, b2, ne)
    return (f.reshape(B, S, N_F), r.reshape(B, S, N_R), r.reshape(B, S, N_R), v.reshape(B, S, N_V))
